# baseline (speedup 1.0000x reference)
import functools

import jax
import jax.numpy as jnp
from jax.experimental import pallas as pl
from jax.experimental.pallas import tpu as pltpu

_BT = 2048


def _fused_kernel(scal_ref, x_ref, g_ref, bet_ref, w_ref, b_ref,
                  out_ref, acc_ref, w2_ref, aux_ref, *, bt, bpb, nb, out_dim):
    i = pl.program_id(0)
    phase1 = i < nb
    j = jnp.where(phase1, i, i - nb)
    b = j // bpb
    start = (j % bpb) * bt
    seqlen = scal_ref[b]
    valid = seqlen > start
    full = seqlen >= start + bt

    @pl.when(i == 0)
    def _init():
        acc_ref[...] = jnp.zeros_like(acc_ref)

    @pl.when(jnp.logical_and(phase1, valid))
    def _stats():
        pos = start + jax.lax.broadcasted_iota(jnp.int32, (1, bt), 1)
        m = (pos < seqlen).astype(jnp.float32)
        x = x_ref[...]
        acc_ref[0:1, :] += jax.lax.dot_general(
            m, x, (((1,), (0,)), ((), ())),
            preferred_element_type=jnp.float32)
        acc_ref[1:2, :] += jax.lax.dot_general(
            m, x * x, (((1,), (0,)), ((), ())),
            preferred_element_type=jnp.float32)
        acc_ref[2:3, :] += jnp.sum(m)

    @pl.when(i == nb - 1)
    def _finalize():
        cnt = jnp.maximum(jnp.max(acc_ref[2:3, :]), 1.0)
        mean = acc_ref[0:1, :] / cnt
        var = acc_ref[1:2, :] / cnt - mean * mean
        scale = jax.lax.rsqrt(var + 1e-5) * g_ref[...][None, :]
        shift = bet_ref[...][None, :] - mean * scale
        w2_ref[...] = w_ref[...] * scale
        brow = b_ref[...][None, :]
        b2 = brow + jax.lax.dot_general(
            shift, w_ref[...], (((1,), (1,)), ((), ())),
            preferred_element_type=jnp.float32)
        bhat = brow * jax.lax.rsqrt(jnp.sum(brow * brow) + 1e-12)
        aux_ref[...] = jnp.concatenate(
            [b2, bhat, jnp.zeros((6, out_dim), jnp.float32)], axis=0)

    phase2 = jnp.logical_not(phase1)

    @pl.when(jnp.logical_and(phase2, full))
    def _apply_full():
        y = jax.lax.dot_general(
            x_ref[...], w2_ref[...], (((1,), (1,)), ((), ())),
            preferred_element_type=jnp.float32) + aux_ref[0:1, :]
        out_ref[...] = y * jax.lax.rsqrt(
            jnp.sum(y * y, axis=1, keepdims=True) + 1e-12)

    @pl.when(jnp.logical_and(phase2, jnp.logical_and(valid, jnp.logical_not(full))))
    def _apply_partial():
        y = jax.lax.dot_general(
            x_ref[...], w2_ref[...], (((1,), (1,)), ((), ())),
            preferred_element_type=jnp.float32) + aux_ref[0:1, :]
        y = y * jax.lax.rsqrt(jnp.sum(y * y, axis=1, keepdims=True) + 1e-12)
        pos = start + jax.lax.broadcasted_iota(jnp.int32, (bt, 1), 0)
        out_ref[...] = jnp.where(pos < seqlen, y, aux_ref[1:2, :])

    @pl.when(jnp.logical_and(phase2, jnp.logical_not(valid)))
    def _apply_pad():
        out_ref[...] = jnp.broadcast_to(aux_ref[1:2, :], (bt, out_dim))


def kernel(payload, seq_lens, gamma, beta, W, b):
    B, T, D = payload.shape
    O = W.shape[0]
    bt = _BT
    bpb = T // bt
    nb = (B * T) // bt

    x2d = payload.reshape(B * T, D)
    seq = seq_lens.astype(jnp.int32)

    starts2d = (jnp.arange(bpb, dtype=jnp.int32) * bt)[None, :]
    valid = (seq[:, None] > starts2d).reshape(nb)
    blk = jnp.arange(nb, dtype=jnp.int32)
    eff = jnp.maximum(jax.lax.cummax(jnp.where(valid, blk, -1)), 0)
    scal = jnp.concatenate([seq, eff.astype(jnp.int32)])

    def _xmap(i, scal):
        return (scal[seq.shape[0] + jnp.where(i < nb, i, i - nb)], 0)

    def _omap(i, scal):
        return (jnp.where(i < nb, 0, i - nb), 0)

    y = pl.pallas_call(
        functools.partial(_fused_kernel, bt=bt, bpb=bpb, nb=nb, out_dim=O),
        grid_spec=pltpu.PrefetchScalarGridSpec(
            num_scalar_prefetch=1,
            grid=(2 * nb,),
            in_specs=[
                pl.BlockSpec((bt, D), _xmap),
                pl.BlockSpec((D,), lambda i, scal: (0,)),
                pl.BlockSpec((D,), lambda i, scal: (0,)),
                pl.BlockSpec((O, D), lambda i, scal: (0, 0)),
                pl.BlockSpec((O,), lambda i, scal: (0,)),
            ],
            out_specs=pl.BlockSpec((bt, O), _omap),
            scratch_shapes=[
                pltpu.VMEM((8, D), jnp.float32),
                pltpu.VMEM((O, D), jnp.float32),
                pltpu.VMEM((8, O), jnp.float32),
            ],
        ),
        out_shape=jax.ShapeDtypeStruct((B * T, O), jnp.float32),
        compiler_params=pltpu.CompilerParams(
            dimension_semantics=("arbitrary",)),
    )(scal, x2d, gamma, beta, W, b)

    return y.reshape(B, T, O)
